# baseline (device time: 360712 ns/iter reference)
import jax
import jax.numpy as jnp
from jax import lax
from jax.experimental import pallas as pl
from jax.experimental.pallas import tpu as pltpu

N_DEV = 16
HOPS = N_DEV // 2
F = 2
SUB = 2


def kernel(x, w_mat):
    m_per, k = x.shape
    _, n_per = w_mat.shape
    m_glob = N_DEV * m_per
    m_frag = m_per // F
    m_sub = m_frag // SUB

    def body(
        x_ref,
        w_ref,
        out_ref,
        r_buf,
        l_buf,
        r_send_sems,
        r_recv_sems,
        l_send_sems,
        l_recv_sems,
        r_credit,
        l_credit,
    ):
        my = lax.axis_index("i")
        left = (my - 1) % N_DEV
        right = (my + 1) % N_DEV

        def gemm_frag(origin, p, chunk_frag):
            out_ref[pl.ds(origin * m_per + p * m_frag, m_frag), :] = jnp.dot(
                chunk_frag, w_ref[...], preferred_element_type=jnp.float32
            )

        def desc(buf, sends, recvs, slot_src, slot_dst, p, s, dev):
            return pltpu.make_async_remote_copy(
                src_ref=buf.at[slot_src, p, pl.ds(s * m_sub, m_sub), :],
                dst_ref=buf.at[slot_dst, p, pl.ds(s * m_sub, m_sub), :],
                send_sem=sends.at[slot_src, p, s],
                recv_sem=recvs.at[slot_dst, p, s],
                device_id=(dev,),
                device_id_type=pl.DeviceIdType.MESH,
            )

        def first_hop_desc(buf, sends, recvs, rs, p, s, dev):
            row0 = p * m_frag + s * m_sub
            return pltpu.make_async_remote_copy(
                src_ref=x_ref.at[pl.ds(row0, m_sub), :],
                dst_ref=buf.at[rs, p, pl.ds(s * m_sub, m_sub), :],
                send_sem=sends.at[rs, p, s],
                recv_sem=recvs.at[rs, p, s],
                device_id=(dev,),
                device_id_type=pl.DeviceIdType.MESH,
            )

        for h in range(HOPS):
            ss = h % 3
            rs = (h + 1) % 3

            if h == 3:
                pl.semaphore_wait(r_credit, 3)
                pl.semaphore_wait(l_credit, 3)
            elif h > 3:
                pl.semaphore_wait(r_credit, 1)
                pl.semaphore_wait(l_credit, 1)

            r_sends = []
            l_sends = []
            for p in range(F):
                for s in range(SUB):
                    if h >= 1:
                        desc(r_buf, r_send_sems, r_recv_sems,
                             ss, ss, p, s, left).wait_recv()
                        desc(l_buf, l_send_sems, l_recv_sems,
                             ss, ss, p, s, right).wait_recv()
                    if h == 0:
                        rd = first_hop_desc(
                            r_buf, r_send_sems, r_recv_sems, rs, p, s, right)
                        ld = first_hop_desc(
                            l_buf, l_send_sems, l_recv_sems, rs, p, s, left)
                        rd.start()
                        ld.start()
                        r_sends.append(rd)
                        l_sends.append(ld)
                    else:
                        if h < HOPS - 1 or p == 0:
                            rd = desc(r_buf, r_send_sems, r_recv_sems,
                                      ss, rs, p, s, right)
                            rd.start()
                            r_sends.append(rd)
                        if h < HOPS - 1 or p == 1:
                            ld = desc(l_buf, l_send_sems, l_recv_sems,
                                      ss, rs, p, s, left)
                            ld.start()
                            l_sends.append(ld)
                if h == 0:
                    gemm_frag(my, p, x_ref[pl.ds(p * m_frag, m_frag), :])
                else:
                    gemm_frag((my - h) % N_DEV, p, r_buf[ss, p, :, :])
                    gemm_frag((my + h) % N_DEV, p, l_buf[ss, p, :, :])

            if h < HOPS - 1:
                pl.semaphore_signal(
                    r_credit,
                    inc=1,
                    device_id=(left,),
                    device_id_type=pl.DeviceIdType.MESH,
                )
                pl.semaphore_signal(
                    l_credit,
                    inc=1,
                    device_id=(right,),
                    device_id_type=pl.DeviceIdType.MESH,
                )

            for rd in r_sends:
                rd.wait_send()
            for ld in l_sends:
                ld.wait_send()

        last = HOPS % 3
        anti = (my + HOPS) % N_DEV
        for s in range(SUB):
            desc(r_buf, r_send_sems, r_recv_sems, last, last, 0, s, left
                 ).wait_recv()
        gemm_frag(anti, 0, r_buf[last, 0, :, :])
        for s in range(SUB):
            desc(l_buf, l_send_sems, l_recv_sems, last, last, 1, s, right
                 ).wait_recv()
        gemm_frag(anti, 1, l_buf[last, 1, :, :])

    return pl.pallas_call(
        body,
        out_shape=jax.ShapeDtypeStruct((m_glob, n_per), jnp.float32),
        in_specs=[
            pl.BlockSpec(memory_space=pltpu.VMEM),
            pl.BlockSpec(memory_space=pltpu.VMEM),
        ],
        out_specs=pl.BlockSpec(memory_space=pltpu.VMEM),
        scratch_shapes=[
            pltpu.VMEM((3, F, m_frag, k), jnp.float32),
            pltpu.VMEM((3, F, m_frag, k), jnp.float32),
            pltpu.SemaphoreType.DMA((3, F, SUB)),
            pltpu.SemaphoreType.DMA((3, F, SUB)),
            pltpu.SemaphoreType.DMA((3, F, SUB)),
            pltpu.SemaphoreType.DMA((3, F, SUB)),
            pltpu.SemaphoreType.REGULAR,
            pltpu.SemaphoreType.REGULAR,
        ],
    )(x, w_mat)


# device time: 354990 ns/iter; 1.0161x vs baseline; 1.0161x over previous
import jax
import jax.numpy as jnp
from jax import lax
from jax.experimental import pallas as pl
from jax.experimental.pallas import tpu as pltpu

N_DEV = 16
HOPS = N_DEV // 2
F = 2
SUB = 2


def kernel(x, w_mat):
    m_per, k = x.shape
    _, n_per = w_mat.shape
    m_glob = N_DEV * m_per
    m_frag = m_per // F
    m_sub = m_frag // SUB

    def body(
        x_ref,
        w_ref,
        out_ref,
        r_buf,
        l_buf,
        r_send_sems,
        r_recv_sems,
        l_send_sems,
        l_recv_sems,
        r_credit,
        l_credit,
    ):
        my = lax.axis_index("i")
        left = (my - 1) % N_DEV
        right = (my + 1) % N_DEV

        barrier_sem = pltpu.get_barrier_semaphore()
        for nbr in (left, right):
            pl.semaphore_signal(
                barrier_sem,
                inc=1,
                device_id=(nbr,),
                device_id_type=pl.DeviceIdType.MESH,
            )
        pl.semaphore_wait(barrier_sem, 2)

        def gemm_frag(origin, p, chunk_frag):
            out_ref[pl.ds(origin * m_per + p * m_frag, m_frag), :] = jnp.dot(
                chunk_frag, w_ref[...], preferred_element_type=jnp.float32
            )

        def desc(buf, sends, recvs, slot_src, slot_dst, p, s, dev):
            return pltpu.make_async_remote_copy(
                src_ref=buf.at[slot_src, p, pl.ds(s * m_sub, m_sub), :],
                dst_ref=buf.at[slot_dst, p, pl.ds(s * m_sub, m_sub), :],
                send_sem=sends.at[slot_src, p, s],
                recv_sem=recvs.at[slot_dst, p, s],
                device_id=(dev,),
                device_id_type=pl.DeviceIdType.MESH,
            )

        def first_hop_desc(buf, sends, recvs, rs, p, s, dev):
            row0 = p * m_frag + s * m_sub
            return pltpu.make_async_remote_copy(
                src_ref=x_ref.at[pl.ds(row0, m_sub), :],
                dst_ref=buf.at[rs, p, pl.ds(s * m_sub, m_sub), :],
                send_sem=sends.at[rs, p, s],
                recv_sem=recvs.at[rs, p, s],
                device_id=(dev,),
                device_id_type=pl.DeviceIdType.MESH,
            )

        for h in range(HOPS):
            ss = h % 3
            rs = (h + 1) % 3

            if h == 3:
                pl.semaphore_wait(r_credit, 3)
                pl.semaphore_wait(l_credit, 3)
            elif h > 3:
                pl.semaphore_wait(r_credit, 1)
                pl.semaphore_wait(l_credit, 1)

            r_sends = []
            l_sends = []
            for p in range(F):
                for s in range(SUB):
                    if h >= 1:
                        desc(r_buf, r_send_sems, r_recv_sems,
                             ss, ss, p, s, left).wait_recv()
                        desc(l_buf, l_send_sems, l_recv_sems,
                             ss, ss, p, s, right).wait_recv()
                    if h == 0:
                        rd = first_hop_desc(
                            r_buf, r_send_sems, r_recv_sems, rs, p, s, right)
                        ld = first_hop_desc(
                            l_buf, l_send_sems, l_recv_sems, rs, p, s, left)
                        rd.start()
                        ld.start()
                        r_sends.append(rd)
                        l_sends.append(ld)
                    else:
                        if h < HOPS - 1 or p == 0:
                            rd = desc(r_buf, r_send_sems, r_recv_sems,
                                      ss, rs, p, s, right)
                            rd.start()
                            r_sends.append(rd)
                        if h < HOPS - 1 or p == 1:
                            ld = desc(l_buf, l_send_sems, l_recv_sems,
                                      ss, rs, p, s, left)
                            ld.start()
                            l_sends.append(ld)
                if h == 0:
                    gemm_frag(my, p, x_ref[pl.ds(p * m_frag, m_frag), :])
                else:
                    gemm_frag((my - h) % N_DEV, p, r_buf[ss, p, :, :])
                    gemm_frag((my + h) % N_DEV, p, l_buf[ss, p, :, :])

            if h < HOPS - 1:
                pl.semaphore_signal(
                    r_credit,
                    inc=1,
                    device_id=(left,),
                    device_id_type=pl.DeviceIdType.MESH,
                )
                pl.semaphore_signal(
                    l_credit,
                    inc=1,
                    device_id=(right,),
                    device_id_type=pl.DeviceIdType.MESH,
                )

            for rd in r_sends:
                rd.wait_send()
            for ld in l_sends:
                ld.wait_send()

        last = HOPS % 3
        anti = (my + HOPS) % N_DEV

        def gemm_sub(p, s, buf):
            row0 = anti * m_per + p * m_frag + s * m_sub
            out_ref[pl.ds(row0, m_sub), :] = jnp.dot(
                buf[last, p, pl.ds(s * m_sub, m_sub), :],
                w_ref[...],
                preferred_element_type=jnp.float32,
            )

        for s in range(SUB):
            desc(r_buf, r_send_sems, r_recv_sems, last, last, 0, s, left
                 ).wait_recv()
            gemm_sub(0, s, r_buf)
            desc(l_buf, l_send_sems, l_recv_sems, last, last, 1, s, right
                 ).wait_recv()
            gemm_sub(1, s, l_buf)

    return pl.pallas_call(
        body,
        out_shape=jax.ShapeDtypeStruct((m_glob, n_per), jnp.float32),
        in_specs=[
            pl.BlockSpec(memory_space=pltpu.VMEM),
            pl.BlockSpec(memory_space=pltpu.VMEM),
        ],
        out_specs=pl.BlockSpec(memory_space=pltpu.VMEM),
        scratch_shapes=[
            pltpu.VMEM((3, F, m_frag, k), jnp.float32),
            pltpu.VMEM((3, F, m_frag, k), jnp.float32),
            pltpu.SemaphoreType.DMA((3, F, SUB)),
            pltpu.SemaphoreType.DMA((3, F, SUB)),
            pltpu.SemaphoreType.DMA((3, F, SUB)),
            pltpu.SemaphoreType.DMA((3, F, SUB)),
            pltpu.SemaphoreType.REGULAR,
            pltpu.SemaphoreType.REGULAR,
        ],
        compiler_params=pltpu.CompilerParams(collective_id=0),
    )(x, w_mat)
